# Initial kernel scaffold; baseline (speedup 1.0000x reference)
import functools

import jax
import jax.numpy as jnp
from jax import lax
from jax.experimental import pallas as pl
from jax.experimental.pallas import tpu as pltpu
from jax.experimental.pallas import tpu_sc as plsc

_N = 10000
_D = 128
_E = 320000
_NC = 2
_NS = 16
_NW = _NC * _NS
_EPW = _E // _NW
_CH = 80
_NCH = _EPW // _CH
_RPT = _N // _NS
_RST = 125
_L = 16


def _lane_bcast(v, l):
  idx = jnp.full((_L, 1), l, jnp.int32)
  dn = lax.GatherDimensionNumbers(
      offset_dims=(), collapsed_slice_dims=(0,), start_index_map=(0,))
  return lax.gather(v, idx, dn, (1,),
                    mode=lax.GatherScatterMode.PROMISE_IN_BOUNDS)


def _sc_body(src_hbm, dst_hbm, vals_hbm, ego_hbm, out_hbm,
             src_v, dst_v, vals_v, gbuf, stage, acc, gsem):
  cid = lax.axis_index("c")
  sid = lax.axis_index("s")
  wid = cid * _NS + sid

  z = jnp.zeros((_L,), jnp.float32)

  def zrow(i, carry):
    for k in range(_D // _L):
      stage[i, pl.ds(k * _L, _L)] = z
    return carry

  lax.fori_loop(0, _RST, zrow, 0)
  for k in range(_RPT // _RST):
    pltpu.sync_copy(stage, acc.at[pl.ds(sid * _RPT + k * _RST, _RST)])
  plsc.subcore_barrier()

  base = wid * _EPW
  pltpu.sync_copy(src_hbm.at[pl.ds(base, _EPW)], src_v)
  pltpu.sync_copy(vals_hbm.at[pl.ds(base, _EPW)], vals_v)
  pltpu.sync_copy(dst_hbm.at[pl.ds(wid * _NCH, _NCH)], dst_v)

  def chunk(c, carry):
    pltpu.async_copy(
        ego_hbm.at[src_v.at[pl.ds(c * _CH, _CH)]], gbuf, gsem).wait()

    def group(g, carry2):
      vv = vals_v[pl.ds(c * _CH + g * _L, _L)]
      e0 = g * _L
      for l in range(_L):
        b = _lane_bcast(vv, l)
        for k in range(_D // _L):
          gbuf[e0 + l, pl.ds(k * _L, _L)] = (
              gbuf[e0 + l, pl.ds(k * _L, _L)] * b)
      return carry2

    lax.fori_loop(0, _CH // _L, group, 0)

    pltpu.sync_copy(gbuf, acc.at[dst_v.at[c]], add=True)
    return carry

  lax.fori_loop(0, _NCH, chunk, 0)
  plsc.subcore_barrier()

  for k in range(_RPT // _RST):
    r0 = sid * _RPT + k * _RST
    pltpu.sync_copy(acc.at[pl.ds(r0, _RST)], stage)
    pltpu.sync_copy(stage, out_hbm.at[pl.ds(cid * _N + r0, _RST)])


def _sc_segment_sum(src, dst2d, vals, ego):
  mesh = plsc.VectorSubcoreMesh(core_axis_name="c", subcore_axis_name="s")
  return pl.kernel(
      _sc_body,
      out_type=jax.ShapeDtypeStruct((_NC * _N, _D), jnp.float32),
      mesh=mesh,
      scratch_types=[
          pltpu.VMEM((_EPW,), jnp.int32),
          pltpu.VMEM((_NCH, _CH), jnp.int32),
          pltpu.VMEM((_EPW,), jnp.float32),
          pltpu.VMEM((_CH, _D), jnp.float32),
          pltpu.VMEM((_RST, _D), jnp.float32),
          pltpu.VMEM_SHARED((_N, _D), jnp.float32),
          pltpu.SemaphoreType.DMA,
      ],
  )(src, dst2d, vals, ego)


_BN = 1000


def _tc_body(p0_ref, p1_ref, ego_ref, wgc_ref, bgc_ref, wbi_ref, bbi_ref,
             out_ref):
  side = p0_ref[...] + p1_ref[...]
  gcn = jnp.dot(side, wgc_ref[...],
                preferred_element_type=jnp.float32) + bgc_ref[...]
  gcn = jnp.where(gcn >= 0, gcn, 0.2 * gcn)
  bi = jnp.dot(ego_ref[...] * side, wbi_ref[...],
               preferred_element_type=jnp.float32) + bbi_ref[...]
  bi = jnp.where(bi >= 0, bi, 0.2 * bi)
  o = gcn + bi
  ss = jnp.sum(o * o, axis=1, keepdims=True)
  out_ref[...] = o / jnp.sqrt(jnp.maximum(ss, 1e-12))


def _tc_mlp(p0, p1, ego, w_gc, b_gc, w_bi, b_bi):
  row_spec = pl.BlockSpec((_BN, _D), lambda i: (i, 0))
  full_w = pl.BlockSpec((_D, _D), lambda i: (0, 0))
  full_b = pl.BlockSpec((1, _D), lambda i: (0, 0))
  return pl.pallas_call(
      _tc_body,
      grid=(_N // _BN,),
      in_specs=[row_spec, row_spec, row_spec, full_w, full_b, full_w, full_b],
      out_specs=row_spec,
      out_shape=jax.ShapeDtypeStruct((_N, _D), jnp.float32),
  )(p0, p1, ego, w_gc, b_gc, w_bi, b_bi)


@jax.jit
def kernel(edge_index, edge_vals, ego_embeddings, w_gc, b_gc, w_bi, b_bi):
  src = edge_index[0].astype(jnp.int32)
  dst2d = edge_index[1].astype(jnp.int32).reshape(_NW * _NCH, _CH)
  partial = _sc_segment_sum(src, dst2d, edge_vals, ego_embeddings)
  return _tc_mlp(partial[:_N], partial[_N:], ego_embeddings,
                 w_gc, b_gc.reshape(1, _D), w_bi, b_bi.reshape(1, _D))


# speedup vs baseline: 6.0203x; 6.0203x over previous
import functools

import jax
import jax.numpy as jnp
from jax import lax
from jax.experimental import pallas as pl
from jax.experimental.pallas import tpu as pltpu
from jax.experimental.pallas import tpu_sc as plsc

_N = 10000
_N2 = 10240
_D = 128
_E = 320000
_NC = 2
_NS = 16
_NW = _NC * _NS
_EPW = _E // _NW
_CH = 80
_SB = 2000
_NSB = _EPW // _SB
_CPS = _SB // _CH
_RPT = _N2 // _NS
_RST = 128
_L = 16


def _lane_bcast(v, l):
  idx = jnp.full((_L, 1), l, jnp.int32)
  dn = lax.GatherDimensionNumbers(
      offset_dims=(), collapsed_slice_dims=(0,), start_index_map=(0,))
  return lax.gather(v, idx, dn, (1,),
                    mode=lax.GatherScatterMode.PROMISE_IN_BOUNDS)


def _sc_body(src_hbm, dst_hbm, vals_hbm, ego_hbm, out_hbm,
             src_v, dst_v, vals_v, gbuf, stage, acc, gsem):
  cid = lax.axis_index("c")
  sid = lax.axis_index("s")
  wid = cid * _NS + sid

  z = jnp.zeros((_L,), jnp.float32)

  def zrow(i, carry):
    for k in range(_D // _L):
      stage[i, pl.ds(k * _L, _L)] = z
    return carry

  lax.fori_loop(0, _RST, zrow, 0)
  for k in range(_RPT // _RST):
    pltpu.sync_copy(stage, acc.at[pl.ds(sid * _RPT + k * _RST, _RST)])
  plsc.subcore_barrier()

  def superblock(sb, carry0):
    base = wid * _EPW + sb * _SB
    pltpu.sync_copy(src_hbm.at[pl.ds(base, _SB)], src_v)
    pltpu.sync_copy(vals_hbm.at[pl.ds(base, _SB)], vals_v)
    pltpu.sync_copy(dst_hbm.at[wid * _NSB + sb], dst_v)

    def chunk(c, carry):
      pltpu.async_copy(
          ego_hbm.at[src_v.at[pl.ds(c * _CH, _CH)]], gbuf, gsem).wait()

      def group(g, carry2):
        vv = vals_v[pl.ds(c * _CH + g * _L, _L)]
        e0 = g * _L
        for l in range(_L):
          b = _lane_bcast(vv, l)
          for k in range(_D // _L):
            gbuf[e0 + l, pl.ds(k * _L, _L)] = (
                gbuf[e0 + l, pl.ds(k * _L, _L)] * b)
        return carry2

      lax.fori_loop(0, _CH // _L, group, 0)

      pltpu.sync_copy(gbuf, acc.at[dst_v.at[c]], add=True)
      return carry

    lax.fori_loop(0, _CPS, chunk, 0)
    return carry0

  lax.fori_loop(0, _NSB, superblock, 0)
  plsc.subcore_barrier()

  for k in range(_RPT // _RST):
    r0 = sid * _RPT + k * _RST
    pltpu.sync_copy(acc.at[pl.ds(r0, _RST)], stage)
    pltpu.sync_copy(stage, out_hbm.at[pl.ds(cid * _N2 + r0, _RST)])


def _sc_segment_sum(src, dst3d, vals, ego):
  mesh = plsc.VectorSubcoreMesh(core_axis_name="c", subcore_axis_name="s")
  return pl.kernel(
      _sc_body,
      out_type=jax.ShapeDtypeStruct((_NC * _N2, _D), jnp.float32),
      mesh=mesh,
      scratch_types=[
          pltpu.VMEM((_SB,), jnp.int32),
          pltpu.VMEM((_CPS, _CH), jnp.int32),
          pltpu.VMEM((_SB,), jnp.float32),
          pltpu.VMEM((_CH, _D), jnp.float32),
          pltpu.VMEM((_RST, _D), jnp.float32),
          pltpu.VMEM_SHARED((_N2, _D), jnp.float32),
          pltpu.SemaphoreType.DMA,
      ],
  )(src, dst3d, vals, ego)


_BN = 1000


def _tc_body(p0_ref, p1_ref, ego_ref, wgc_ref, bgc_ref, wbi_ref, bbi_ref,
             out_ref):
  side = p0_ref[...] + p1_ref[...]
  gcn = jnp.dot(side, wgc_ref[...],
                preferred_element_type=jnp.float32) + bgc_ref[...]
  gcn = jnp.where(gcn >= 0, gcn, 0.2 * gcn)
  bi = jnp.dot(ego_ref[...] * side, wbi_ref[...],
               preferred_element_type=jnp.float32) + bbi_ref[...]
  bi = jnp.where(bi >= 0, bi, 0.2 * bi)
  o = gcn + bi
  ss = jnp.sum(o * o, axis=1, keepdims=True)
  out_ref[...] = o / jnp.sqrt(jnp.maximum(ss, 1e-12))


def _tc_mlp(p0, p1, ego, w_gc, b_gc, w_bi, b_bi):
  row_spec = pl.BlockSpec((_BN, _D), lambda i: (i, 0))
  full_w = pl.BlockSpec((_D, _D), lambda i: (0, 0))
  full_b = pl.BlockSpec((1, _D), lambda i: (0, 0))
  return pl.pallas_call(
      _tc_body,
      grid=(_N // _BN,),
      in_specs=[row_spec, row_spec, row_spec, full_w, full_b, full_w, full_b],
      out_specs=row_spec,
      out_shape=jax.ShapeDtypeStruct((_N, _D), jnp.float32),
  )(p0, p1, ego, w_gc, b_gc, w_bi, b_bi)


@jax.jit
def kernel(edge_index, edge_vals, ego_embeddings, w_gc, b_gc, w_bi, b_bi):
  src = edge_index[0].astype(jnp.int32)
  dst3d = edge_index[1].astype(jnp.int32).reshape(_NW * _NSB, _CPS, _CH)
  partial = _sc_segment_sum(src, dst3d, edge_vals, ego_embeddings)
  return _tc_mlp(partial[:_N], partial[_N2:_N2 + _N], ego_embeddings,
                 w_gc, b_gc.reshape(1, _D), w_bi, b_bi.reshape(1, _D))


# speedup vs baseline: 8.7820x; 1.4587x over previous
import functools

import jax
import jax.numpy as jnp
from jax import lax
from jax.experimental import pallas as pl
from jax.experimental.pallas import tpu as pltpu
from jax.experimental.pallas import tpu_sc as plsc

_N = 10000
_N2 = 10240
_D = 128
_E = 320000
_NC = 2
_NS = 16
_NW = _NC * _NS
_EPW = _E // _NW
_CH = 80
_SB = 2000
_NSB = _EPW // _SB
_CPS = _SB // _CH
_RPT = _N2 // _NS
_L = 16


def _lane_bcast(v, l):
  idx = jnp.full((_L, 1), l, jnp.int32)
  dn = lax.GatherDimensionNumbers(
      offset_dims=(), collapsed_slice_dims=(0,), start_index_map=(0,))
  return lax.gather(v, idx, dn, (1,),
                    mode=lax.GatherScatterMode.PROMISE_IN_BOUNDS)


def _sc_body(src_hbm, dst_hbm, vals_hbm, ego_hbm, out0_hbm, out1_hbm,
             src_v, dst_v, vals_v, gb0, gb1, acc, gs0, gs1, ss0, ss1):
  cid = lax.axis_index("c")
  sid = lax.axis_index("s")
  wid = cid * _NS + sid

  def start_gather(c, buf, sem):
    return pltpu.async_copy(
        ego_hbm.at[src_v.at[pl.ds(c * _CH, _CH)]], buf, sem)

  def wait_gather(c, buf, sem):
    pltpu.make_async_copy(
        ego_hbm.at[src_v.at[pl.ds(c * _CH, _CH)]], buf, sem).wait()

  def start_scatter(c, buf, sem):
    return pltpu.async_copy(buf, acc.at[dst_v.at[c]], sem, add=True)

  def wait_scatter(c, buf, sem):
    pltpu.make_async_copy(buf, acc.at[dst_v.at[c]], sem).wait()

  def scale(buf, c):
    def group(g, carry):
      vv = vals_v[pl.ds(c * _CH + g * _L, _L)]
      e0 = g * _L
      for l in range(_L):
        b = _lane_bcast(vv, l)
        for k in range(_D // _L):
          buf[e0 + l, pl.ds(k * _L, _L)] = buf[e0 + l, pl.ds(k * _L, _L)] * b
      return carry

    lax.fori_loop(0, _CH // _L, group, 0)

  z = jnp.zeros((_L,), jnp.float32)

  def zrow(i, carry):
    for k in range(_D // _L):
      gb0[i, pl.ds(k * _L, _L)] = z
    return carry

  lax.fori_loop(0, _CH, zrow, 0)
  for k in range(_RPT // _CH):
    pltpu.sync_copy(gb0, acc.at[pl.ds(sid * _RPT + k * _CH, _CH)])
  plsc.subcore_barrier()

  def superblock(sb, carry0):
    base = wid * _EPW + sb * _SB
    pltpu.sync_copy(src_hbm.at[pl.ds(base, _SB)], src_v)
    pltpu.sync_copy(vals_hbm.at[pl.ds(base, _SB)], vals_v)
    pltpu.sync_copy(dst_hbm.at[wid * _NSB + sb], dst_v)

    start_gather(0, gb0, gs0)
    start_gather(1, gb1, gs1)

    def pair(pp, carry):
      c0 = 2 * pp
      c1 = c0 + 1
      wait_gather(c0, gb0, gs0)
      scale(gb0, c0)
      start_scatter(c0, gb0, ss0)
      wait_gather(c1, gb1, gs1)
      scale(gb1, c1)
      start_scatter(c1, gb1, ss1)
      wait_scatter(c0, gb0, ss0)
      start_gather(c0 + 2, gb0, gs0)

      @pl.when(pp < _CPS // 2 - 1)
      def _():
        wait_scatter(c1, gb1, ss1)
        start_gather(c1 + 2, gb1, gs1)

      return carry

    lax.fori_loop(0, _CPS // 2, pair, 0)
    wait_scatter(_CPS - 2, gb1, ss1)
    wait_gather(_CPS - 1, gb0, gs0)
    scale(gb0, _CPS - 1)
    start_scatter(_CPS - 1, gb0, ss0)
    wait_scatter(_CPS - 1, gb0, ss0)
    return carry0

  lax.fori_loop(0, _NSB, superblock, 0)
  plsc.subcore_barrier()

  def dump(oref):
    for k in range(_RPT // _CH):
      r0 = sid * _RPT + k * _CH
      pltpu.sync_copy(acc.at[pl.ds(r0, _CH)], gb0)
      pltpu.sync_copy(gb0, oref.at[pl.ds(r0, _CH)])

  @pl.when(cid == 0)
  def _():
    dump(out0_hbm)

  @pl.when(cid == 1)
  def _():
    dump(out1_hbm)


def _sc_segment_sum(src, dst3d, vals, ego):
  mesh = plsc.VectorSubcoreMesh(core_axis_name="c", subcore_axis_name="s")
  return pl.kernel(
      _sc_body,
      out_type=(jax.ShapeDtypeStruct((_N2, _D), jnp.float32),
                jax.ShapeDtypeStruct((_N2, _D), jnp.float32)),
      mesh=mesh,
      scratch_types=[
          pltpu.VMEM((_SB,), jnp.int32),
          pltpu.VMEM((_CPS, _CH), jnp.int32),
          pltpu.VMEM((_SB,), jnp.float32),
          pltpu.VMEM((_CH, _D), jnp.float32),
          pltpu.VMEM((_CH, _D), jnp.float32),
          pltpu.VMEM_SHARED((_N2, _D), jnp.float32),
          pltpu.SemaphoreType.DMA,
          pltpu.SemaphoreType.DMA,
          pltpu.SemaphoreType.DMA,
          pltpu.SemaphoreType.DMA,
      ],
  )(src, dst3d, vals, ego)


_BN = 1000


def _tc_body(p0_ref, p1_ref, ego_ref, wgc_ref, bgc_ref, wbi_ref, bbi_ref,
             out_ref):
  side = p0_ref[...] + p1_ref[...]
  gcn = jnp.dot(side, wgc_ref[...],
                preferred_element_type=jnp.float32) + bgc_ref[...]
  gcn = jnp.where(gcn >= 0, gcn, 0.2 * gcn)
  bi = jnp.dot(ego_ref[...] * side, wbi_ref[...],
               preferred_element_type=jnp.float32) + bbi_ref[...]
  bi = jnp.where(bi >= 0, bi, 0.2 * bi)
  o = gcn + bi
  ss = jnp.sum(o * o, axis=1, keepdims=True)
  out_ref[...] = o / jnp.sqrt(jnp.maximum(ss, 1e-12))


def _tc_mlp(p0, p1, ego, w_gc, b_gc, w_bi, b_bi):
  row_spec = pl.BlockSpec((_BN, _D), lambda i: (i, 0))
  full_w = pl.BlockSpec((_D, _D), lambda i: (0, 0))
  full_b = pl.BlockSpec((1, _D), lambda i: (0, 0))
  return pl.pallas_call(
      _tc_body,
      grid=(_N // _BN,),
      in_specs=[row_spec, row_spec, row_spec, full_w, full_b, full_w, full_b],
      out_specs=row_spec,
      out_shape=jax.ShapeDtypeStruct((_N, _D), jnp.float32),
  )(p0, p1, ego, w_gc, b_gc, w_bi, b_bi)


@jax.jit
def kernel(edge_index, edge_vals, ego_embeddings, w_gc, b_gc, w_bi, b_bi):
  src = edge_index[0].astype(jnp.int32)
  dst3d = edge_index[1].astype(jnp.int32).reshape(_NW * _NSB, _CPS, _CH)
  p0, p1 = _sc_segment_sum(src, dst3d, edge_vals, ego_embeddings)
  return _tc_mlp(p0, p1, ego_embeddings,
                 w_gc, b_gc.reshape(1, _D), w_bi, b_bi.reshape(1, _D))
